# Initial kernel scaffold; baseline (speedup 1.0000x reference)
import jax
import jax.numpy as jnp
from jax.experimental import pallas as pl


def kernel(word_embed_idx, pre_tag_embed, table, W1, b1, W2, b2):
    raise NotImplementedError("write your pallas kernel here")


# speedup vs baseline: 1.7218x; 1.7218x over previous
import jax
import jax.numpy as jnp
from jax import lax
from jax.experimental import pallas as pl
from jax.experimental.pallas import tpu as pltpu


def _body(idx_ref, row_ref, tag_ref, w1_ref, b1_ref, w2_ref, b2_ref, out_ref):
    del idx_ref
    row = row_ref[0]
    tag = tag_ref[...]
    cat = jnp.concatenate([row, tag], axis=1)
    z1 = lax.dot_general(
        cat, w1_ref[...], (((1,), (1,)), ((), ())),
        preferred_element_type=jnp.float32,
    ) + b1_ref[...]
    a1 = jnp.maximum(z1, 0.0)
    z2 = lax.dot_general(
        a1, w2_ref[...], (((1,), (1,)), ((), ())),
        preferred_element_type=jnp.float32,
    ) + b2_ref[...]
    m = jnp.max(z2, axis=1, keepdims=True)
    s = jnp.sum(jnp.exp(z2 - m), axis=1, keepdims=True)
    out_ref[...] = z2 - m - jnp.log(s)


@jax.jit
def kernel(word_embed_idx, pre_tag_embed, table, W1, b1, W2, b2):
    idx = word_embed_idx.astype(jnp.int32)
    grid_spec = pltpu.PrefetchScalarGridSpec(
        num_scalar_prefetch=1,
        grid=(1,),
        in_specs=[
            pl.BlockSpec((1, 1, 128), lambda i, idx_ref: (idx_ref[0], 0, 0)),
            pl.BlockSpec((1, 64), lambda i, idx_ref: (0, 0)),
            pl.BlockSpec((256, 192), lambda i, idx_ref: (0, 0)),
            pl.BlockSpec((1, 256), lambda i, idx_ref: (0, 0)),
            pl.BlockSpec((64, 256), lambda i, idx_ref: (0, 0)),
            pl.BlockSpec((1, 64), lambda i, idx_ref: (0, 0)),
        ],
        out_specs=pl.BlockSpec((1, 64), lambda i, idx_ref: (0, 0)),
    )
    return pl.pallas_call(
        _body,
        grid_spec=grid_spec,
        out_shape=jax.ShapeDtypeStruct((1, 64), jnp.float32),
    )(idx, table.reshape(-1, 1, 128), pre_tag_embed, W1,
      b1.reshape(1, -1), W2, b2.reshape(1, -1))
